# Initial kernel scaffold; baseline (speedup 1.0000x reference)
import jax
import jax.numpy as jnp
from jax.experimental import pallas as pl


def kernel(x, edge_index, W1_0, W1_1, b1, W2_0, W2_1, b2):
    raise NotImplementedError("write your pallas kernel here")


# speedup vs baseline: 16.9078x; 16.9078x over previous
import functools

import jax
import jax.numpy as jnp
from jax import lax
from jax.experimental import pallas as pl
from jax.experimental.pallas import tpu as pltpu
from jax.experimental.pallas import tpu_sc as plsc

N_NODES = 10000
N_EDGES = 160000
D_IN = 256
HID = 16
NCLS = 64

CHUNK = 128
NW = 32
CH_PER_TILE = 40
E_PAD = NW * CH_PER_TILE * CHUNK
CH_TOT = E_PAD // CHUNK
ACC_ROWS = 10240
ROWS_PER_TILE = ACC_ROWS // 16
TAB_ROWS = N_NODES + 16

_MESH = plsc.VectorSubcoreMesh(core_axis_name="c", subcore_axis_name="s")
_SC_PARAMS = pltpu.CompilerParams(use_tc_tiling_on_sc=False)


@functools.partial(
    pl.kernel,
    out_type=jax.ShapeDtypeStruct((2, ACC_ROWS, HID), jnp.float32),
    mesh=_MESH,
    scratch_types=[
        pltpu.VMEM((CH_PER_TILE, CHUNK), jnp.int32),
        pltpu.VMEM((CHUNK, HID), jnp.float32),
        pltpu.VMEM_SHARED((ACC_ROWS, HID), jnp.float32),
    ],
    compiler_params=_SC_PARAMS,
)
def _sc_degree(row_hbm, zeros_hbm, ones_hbm, out_hbm, idx_v, ones_v, acc_sh):
    c = lax.axis_index("c")
    s = lax.axis_index("s")
    wid = s * 2 + c
    pltpu.sync_copy(row_hbm.at[pl.ds(wid * CH_PER_TILE, CH_PER_TILE)], idx_v)
    pltpu.sync_copy(ones_hbm, ones_v)
    pltpu.sync_copy(zeros_hbm.at[pl.ds(s * ROWS_PER_TILE, ROWS_PER_TILE)],
                    acc_sh.at[pl.ds(s * ROWS_PER_TILE, ROWS_PER_TILE)])
    plsc.subcore_barrier()

    def body(j, carry):
        pltpu.sync_copy(ones_v, acc_sh.at[idx_v.at[j]], add=True)
        return carry

    lax.fori_loop(0, CH_PER_TILE, body, 0)
    plsc.subcore_barrier()
    pltpu.sync_copy(acc_sh.at[pl.ds(s * ROWS_PER_TILE, ROWS_PER_TILE)],
                    out_hbm.at[c].at[pl.ds(s * ROWS_PER_TILE, ROWS_PER_TILE)])


@functools.partial(
    pl.kernel,
    out_type=jax.ShapeDtypeStruct((2, ACC_ROWS, HID), jnp.float32),
    mesh=_MESH,
    scratch_types=[
        pltpu.VMEM((CH_PER_TILE, CHUNK), jnp.int32),
        pltpu.VMEM((CH_PER_TILE, CHUNK), jnp.int32),
        pltpu.VMEM((CHUNK, HID), jnp.float32),
        pltpu.VMEM_SHARED((ACC_ROWS, HID), jnp.float32),
        pltpu.SemaphoreType.DMA,
    ],
    compiler_params=_SC_PARAMS,
)
def _sc_edge_pass(row_hbm, col_hbm, tab_hbm, zeros_hbm, out_hbm,
                  rowi_v, coli_v, rows_v, acc_sh, sem):
    c = lax.axis_index("c")
    s = lax.axis_index("s")
    wid = s * 2 + c
    pltpu.sync_copy(row_hbm.at[pl.ds(wid * CH_PER_TILE, CH_PER_TILE)], rowi_v)
    pltpu.sync_copy(col_hbm.at[pl.ds(wid * CH_PER_TILE, CH_PER_TILE)], coli_v)
    pltpu.sync_copy(zeros_hbm.at[pl.ds(s * ROWS_PER_TILE, ROWS_PER_TILE)],
                    acc_sh.at[pl.ds(s * ROWS_PER_TILE, ROWS_PER_TILE)])
    plsc.subcore_barrier()

    def body(j, carry):
        pltpu.async_copy(tab_hbm.at[rowi_v.at[j]], rows_v, sem).wait()
        pltpu.sync_copy(rows_v, acc_sh.at[coli_v.at[j]], add=True)
        return carry

    lax.fori_loop(0, CH_PER_TILE, body, 0)
    plsc.subcore_barrier()
    pltpu.sync_copy(acc_sh.at[pl.ds(s * ROWS_PER_TILE, ROWS_PER_TILE)],
                    out_hbm.at[c].at[pl.ds(s * ROWS_PER_TILE, ROWS_PER_TILE)])


_BLK = 1000


def _tc_dense1_body(x_ref, w_ref, d0_ref, d1_ref, b1_ref,
                    a_ref, yt_ref, dinv_ref):
    deg = d0_ref[...] + d1_ref[...]
    dinv = jnp.where(deg > 0, lax.rsqrt(jnp.where(deg > 0, deg, 1.0)), 0.0)
    u = jnp.dot(x_ref[...], w_ref[...], preferred_element_type=jnp.float32)
    a_ref[...] = u[:, :HID] + b1_ref[...]
    yt_ref[...] = dinv * u[:, HID:]
    dinv_ref[...] = dinv


def _tc_dense1(x, wcat, d0, d1, b1):
    grid = (N_NODES // _BLK,)
    return pl.pallas_call(
        _tc_dense1_body,
        grid=grid,
        in_specs=[
            pl.BlockSpec((_BLK, D_IN), lambda i: (i, 0)),
            pl.BlockSpec((D_IN, 2 * HID), lambda i: (0, 0)),
            pl.BlockSpec((_BLK, HID), lambda i: (i, 0)),
            pl.BlockSpec((_BLK, HID), lambda i: (i, 0)),
            pl.BlockSpec((1, HID), lambda i: (0, 0)),
        ],
        out_specs=[
            pl.BlockSpec((_BLK, HID), lambda i: (i, 0)),
            pl.BlockSpec((_BLK, HID), lambda i: (i, 0)),
            pl.BlockSpec((_BLK, HID), lambda i: (i, 0)),
        ],
        out_shape=[
            jax.ShapeDtypeStruct((N_NODES, HID), jnp.float32),
            jax.ShapeDtypeStruct((N_NODES, HID), jnp.float32),
            jax.ShapeDtypeStruct((N_NODES, HID), jnp.float32),
        ],
    )(x, wcat, d0, d1, b1)


def _tc_mid_body(a_ref, s0_ref, s1_ref, dinv_ref, h_ref, ht_ref):
    dinv = dinv_ref[...]
    h = jnp.maximum(a_ref[...] - dinv * (s0_ref[...] + s1_ref[...]), 0.0)
    h_ref[...] = h
    ht_ref[...] = dinv * h


def _tc_mid(a, s0, s1, dinv):
    grid = (N_NODES // _BLK,)
    spec = pl.BlockSpec((_BLK, HID), lambda i: (i, 0))
    return pl.pallas_call(
        _tc_mid_body,
        grid=grid,
        in_specs=[spec, spec, spec, spec],
        out_specs=[spec, spec],
        out_shape=[
            jax.ShapeDtypeStruct((N_NODES, HID), jnp.float32),
            jax.ShapeDtypeStruct((N_NODES, HID), jnp.float32),
        ],
    )(a, s0, s1, dinv)


def _tc_out_body(h_ref, t0_ref, t1_ref, dinv_ref, w0_ref, w1_ref, b2_ref,
                 o_ref):
    tt = dinv_ref[...] * (t0_ref[...] + t1_ref[...])
    z = (jnp.dot(h_ref[...], w0_ref[...], preferred_element_type=jnp.float32)
         - jnp.dot(tt, w1_ref[...], preferred_element_type=jnp.float32)
         + b2_ref[...])
    m = jnp.max(z, axis=1, keepdims=True)
    e = jnp.exp(z - m)
    o_ref[...] = z - m - jnp.log(jnp.sum(e, axis=1, keepdims=True))


def _tc_out(h, t0, t1, dinv, w20, w21, b2):
    grid = (N_NODES // _BLK,)
    spec16 = pl.BlockSpec((_BLK, HID), lambda i: (i, 0))
    return pl.pallas_call(
        _tc_out_body,
        grid=grid,
        in_specs=[
            spec16, spec16, spec16, spec16,
            pl.BlockSpec((HID, NCLS), lambda i: (0, 0)),
            pl.BlockSpec((HID, NCLS), lambda i: (0, 0)),
            pl.BlockSpec((1, NCLS), lambda i: (0, 0)),
        ],
        out_specs=pl.BlockSpec((_BLK, NCLS), lambda i: (i, 0)),
        out_shape=jax.ShapeDtypeStruct((N_NODES, NCLS), jnp.float32),
    )(h, t0, t1, dinv, w20, w21, b2)


def kernel(x, edge_index, W1_0, W1_1, b1, W2_0, W2_1, b2):
    row = edge_index[0]
    col = edge_index[1]
    pad = jnp.full((E_PAD - N_EDGES,), N_NODES, dtype=jnp.int32)
    row2d = jnp.concatenate([row, pad]).reshape(CH_TOT, CHUNK)
    col2d = jnp.concatenate([col, pad]).reshape(CH_TOT, CHUNK)
    zeros_acc = jnp.zeros((ACC_ROWS, HID), jnp.float32)
    ones_src = jnp.ones((CHUNK, HID), jnp.float32)
    tab_pad = jnp.zeros((TAB_ROWS - N_NODES, HID), jnp.float32)

    d = _sc_degree(row2d, zeros_acc, ones_src)
    wcat = jnp.concatenate([W1_0, W1_1], axis=1)
    a, yt, dinv = _tc_dense1(x, wcat, d[0, :N_NODES], d[1, :N_NODES],
                             b1.reshape(1, HID))

    s = _sc_edge_pass(row2d, col2d, jnp.concatenate([yt, tab_pad]), zeros_acc)
    h, ht = _tc_mid(a, s[0, :N_NODES], s[1, :N_NODES], dinv)

    t = _sc_edge_pass(row2d, col2d, jnp.concatenate([ht, tab_pad]), zeros_acc)
    return _tc_out(h, t[0, :N_NODES], t[1, :N_NODES], dinv, W2_0, W2_1,
                   b2.reshape(1, NCLS))
